# baseline (speedup 1.0000x reference)
import functools

import jax
import jax.numpy as jnp
from jax import lax
from jax.experimental import pallas as pl
from jax.experimental.pallas import tpu as pltpu
from jax.experimental.pallas import tpu_sc as plsc

NS = 16
B = 80
RING = 4
CH = 16
ROWS = 1000


def _deg_body(eidx_h, out_h, hist, idxb, mbuf, res, shist):
    cid = lax.axis_index("c")
    sid = lax.axis_index("s")
    n_pad = shist.shape[1]
    npt = n_pad // NS
    e = eidx_h.shape[0] // 2
    ept = e // NS
    zero16 = jnp.zeros((16,), jnp.float32)
    ones16 = jnp.full((16,), 1.0, jnp.float32)

    def z_step(i, c):
        hist[pl.ds(i * 16, 16)] = zero16
        return c

    lax.fori_loop(0, n_pad // 16, z_step, 0)
    pltpu.sync_copy(eidx_h.at[pl.ds(cid * e + sid * ept, ept)], idxb)

    def h_step(i, c):
        iv = idxb[pl.ds(i * 16, 16)]
        plsc.addupdate_scatter(hist, [iv], ones16)
        return c

    lax.fori_loop(0, ept // 16, h_step, 0)
    pltpu.sync_copy(hist, shist.at[sid])
    plsc.subcore_barrier()

    for k in range(NS):
        pltpu.sync_copy(shist.at[k, pl.ds(sid * npt, npt)], mbuf.at[k])

    def r_step(i, c):
        s = mbuf[0, pl.ds(i * 16, 16)]
        for k in range(1, NS):
            s = s + mbuf[k, pl.ds(i * 16, 16)]
        res[pl.ds(i * 16, 16)] = s
        return c

    lax.fori_loop(0, npt // 16, r_step, 0)
    pltpu.sync_copy(res, out_h.at[cid, pl.ds(sid * npt, npt)])


def _agg_body(hsd_h, gidx_h, sidx_h, z_h, out_h, acc,
              gb0, gb1, gb2, gb3, gidxb, sidxb, sm0, sm1, sm2, sm3):
    cid = lax.axis_index("c")
    sid = lax.axis_index("s")
    n_pad = acc.shape[0]
    npt = n_pad // NS
    gbufs = [gb0, gb1, gb2, gb3]
    sems = [sm0, sm1, sm2, sm3]
    nbt = gidx_h.shape[0] // (2 * NS)
    pltpu.sync_copy(z_h, acc.at[pl.ds(sid * npt, npt)])
    roff = (cid * NS + sid) * nbt
    plsc.subcore_barrier()

    def chunk(c, carry):
        off = roff + c * CH
        pltpu.sync_copy(gidx_h.at[pl.ds(off, CH)], gidxb)
        pltpu.sync_copy(sidx_h.at[pl.ds(off, CH)], sidxb)
        for b in range(RING):
            pltpu.async_copy(hsd_h.at[gidxb.at[b]], gbufs[b], sems[b])

        def step(i, c2):
            for b in range(RING):
                bi = i * RING + b
                pltpu.make_async_copy(hsd_h.at[gidxb.at[bi]], gbufs[b],
                                      sems[b]).wait()
                pltpu.sync_copy(gbufs[b], acc.at[sidxb.at[bi]], add=True)
                pltpu.async_copy(hsd_h.at[gidxb.at[bi + RING]], gbufs[b],
                                 sems[b])
            return c2

        lax.fori_loop(0, CH // RING - 1, step, 0)
        for b in range(RING):
            bi = CH - RING + b
            pltpu.make_async_copy(hsd_h.at[gidxb.at[bi]], gbufs[b],
                                  sems[b]).wait()
            pltpu.sync_copy(gbufs[b], acc.at[sidxb.at[bi]], add=True)
        return carry

    lax.fori_loop(0, nbt // CH, chunk, 0)
    plsc.subcore_barrier()
    pltpu.sync_copy(acc.at[pl.ds(sid * npt, npt)],
                    out_h.at[cid, pl.ds(sid * npt, npt)])


def _inv_sqrt(deg):
    return jnp.where(deg > 0.0, lax.rsqrt(deg), 0.0)


def _pre_body(x_ref, w_ref, b_ref, deg_ref, h_ref, hsd_ref):
    h = jnp.maximum(
        jnp.dot(x_ref[...], w_ref[...], preferred_element_type=jnp.float32)
        + b_ref[...], 0.0)
    iinv = _inv_sqrt(deg_ref[1, :, 0:1])
    oinv = _inv_sqrt(deg_ref[0, :, 0:1])
    h_ref[...] = h
    hsd_ref[0] = iinv * h
    hsd_ref[1] = oinv * h


def _post_body(agg_ref, deg_ref, h_ref, ws_ref, bs_ref, wd_ref, bd_ref,
               wl_ref, bl_ref, out_ref):
    oinv = _inv_sqrt(deg_ref[0, :, 0:1])
    iinv = _inv_sqrt(deg_ref[1, :, 0:1])
    aggf = agg_ref[0] * oinv
    aggb = agg_ref[1] * iinv
    conv = (0.5 * (jnp.dot(aggf, ws_ref[...],
                           preferred_element_type=jnp.float32) + bs_ref[...])
            + 0.5 * (jnp.dot(aggb, wd_ref[...],
                             preferred_element_type=jnp.float32) + bd_ref[...]))
    h2 = jnp.maximum(conv + h_ref[...], 0.0)
    out_ref[...] = (jnp.dot(h2, wl_ref[...],
                            preferred_element_type=jnp.float32) + bl_ref[...])


@functools.partial(jax.jit, static_argnums=())
def kernel(x, edge_index, W_pre, b_pre, W_s2d, b_s2d, W_d2s, b_d2s,
           W_lin, b_lin):
    n, d = x.shape
    e = edge_index.shape[1]
    npt = (-(-n // NS) + 15) // 16 * 16
    n_pad = npt * NS
    ept = e // NS
    eidx = edge_index.astype(jnp.int32)
    row = eidx[0]
    col = eidx[1]
    gidx = jnp.concatenate([col, row + n])
    sidx = eidx.reshape(2 * e)
    npta = (-(-(n + 1) // NS) + 7) // 8 * 8
    n_pada = npta * NS
    zeros = jnp.zeros((npta, d), jnp.float32)

    mesh = plsc.VectorSubcoreMesh(core_axis_name="c", subcore_axis_name="s")

    deg2 = pl.kernel(
        _deg_body,
        out_type=jax.ShapeDtypeStruct((2, n_pad), jnp.float32),
        mesh=mesh,
        compiler_params=pltpu.CompilerParams(needs_layout_passes=False),
        scratch_types=[
            pltpu.VMEM((n_pad,), jnp.float32),
            pltpu.VMEM((ept,), jnp.int32),
            pltpu.VMEM((NS, npt), jnp.float32),
            pltpu.VMEM((npt,), jnp.float32),
            pltpu.VMEM_SHARED((NS, n_pad), jnp.float32),
        ],
    )(sidx)
    deg3 = deg2[:, :n].reshape(2, n, 1)

    grid = n // ROWS
    h, hsd = pl.pallas_call(
        _pre_body,
        grid=(grid,),
        in_specs=[
            pl.BlockSpec((ROWS, d), lambda i: (i, 0)),
            pl.BlockSpec((d, d), lambda i: (0, 0)),
            pl.BlockSpec((1, d), lambda i: (0, 0)),
            pl.BlockSpec((2, ROWS, 1), lambda i: (0, i, 0)),
        ],
        out_specs=[
            pl.BlockSpec((ROWS, d), lambda i: (i, 0)),
            pl.BlockSpec((2, ROWS, d), lambda i: (0, i, 0)),
        ],
        out_shape=[
            jax.ShapeDtypeStruct((n, d), jnp.float32),
            jax.ShapeDtypeStruct((2, n, d), jnp.float32),
        ],
    )(x, W_pre, b_pre.reshape(1, d), deg3)

    nbt = -(-ept // B)
    nbt = (nbt + CH - 1) // CH * CH
    epp = nbt * B
    gidx2 = jnp.pad(gidx.reshape(2, NS, ept),
                    ((0, 0), (0, 0), (0, epp - ept))).reshape(2 * NS * nbt, B)
    sidx2 = jnp.pad(sidx.reshape(2, NS, ept),
                    ((0, 0), (0, 0), (0, epp - ept)),
                    constant_values=n_pada - 1).reshape(2 * NS * nbt, B)

    agg2 = pl.kernel(
        _agg_body,
        out_type=jax.ShapeDtypeStruct((2, n_pada, d), jnp.float32),
        mesh=mesh,
        scratch_types=[
            pltpu.VMEM_SHARED((n_pada, d), jnp.float32),
            pltpu.VMEM((B, d), jnp.float32),
            pltpu.VMEM((B, d), jnp.float32),
            pltpu.VMEM((B, d), jnp.float32),
            pltpu.VMEM((B, d), jnp.float32),
            pltpu.VMEM((CH, B), jnp.int32),
            pltpu.VMEM((CH, B), jnp.int32),
            pltpu.SemaphoreType.DMA,
            pltpu.SemaphoreType.DMA,
            pltpu.SemaphoreType.DMA,
            pltpu.SemaphoreType.DMA,
        ],
    )(hsd.reshape(2 * n, d), gidx2, sidx2, zeros)

    out = pl.pallas_call(
        _post_body,
        grid=(grid,),
        in_specs=[
            pl.BlockSpec((2, ROWS, d), lambda i: (0, i, 0)),
            pl.BlockSpec((2, ROWS, 1), lambda i: (0, i, 0)),
            pl.BlockSpec((ROWS, d), lambda i: (i, 0)),
            pl.BlockSpec((d, d), lambda i: (0, 0)),
            pl.BlockSpec((1, d), lambda i: (0, 0)),
            pl.BlockSpec((d, d), lambda i: (0, 0)),
            pl.BlockSpec((1, d), lambda i: (0, 0)),
            pl.BlockSpec((d, d), lambda i: (0, 0)),
            pl.BlockSpec((1, d), lambda i: (0, 0)),
        ],
        out_specs=pl.BlockSpec((ROWS, d), lambda i: (i, 0)),
        out_shape=jax.ShapeDtypeStruct((n, d), jnp.float32),
    )(agg2, deg3, h, W_s2d, b_s2d.reshape(1, d), W_d2s, b_d2s.reshape(1, d),
      W_lin, b_lin.reshape(1, d))
    return out


# speedup vs baseline: 1.0454x; 1.0454x over previous
import functools

import jax
import jax.numpy as jnp
from jax import lax
from jax.experimental import pallas as pl
from jax.experimental.pallas import tpu as pltpu
from jax.experimental.pallas import tpu_sc as plsc

NS = 16
B = 80
RING = 4
CH = 16
ROWS = 1000


def _deg_body(eidx_h, out_h, hist, idxb, mbuf, res, shist):
    cid = lax.axis_index("c")
    sid = lax.axis_index("s")
    n_pad = shist.shape[1]
    npt = n_pad // NS
    e = eidx_h.shape[0] // 2
    ept = e // NS
    zero16 = jnp.zeros((16,), jnp.float32)
    ones16 = jnp.full((16,), 1.0, jnp.float32)

    def z_step(i, c):
        hist[pl.ds(i * 16, 16)] = zero16
        return c

    lax.fori_loop(0, n_pad // 16, z_step, 0)
    pltpu.sync_copy(eidx_h.at[pl.ds(cid * e + sid * ept, ept)], idxb)

    def h_step(i, c):
        iv = idxb[pl.ds(i * 16, 16)]
        plsc.addupdate_scatter(hist, [iv], ones16)
        return c

    lax.fori_loop(0, ept // 16, h_step, 0)
    pltpu.sync_copy(hist, shist.at[sid])
    plsc.subcore_barrier()

    for k in range(NS):
        pltpu.sync_copy(shist.at[k, pl.ds(sid * npt, npt)], mbuf.at[k])

    def r_step(i, c):
        s = mbuf[0, pl.ds(i * 16, 16)]
        for k in range(1, NS):
            s = s + mbuf[k, pl.ds(i * 16, 16)]
        res[pl.ds(i * 16, 16)] = s
        return c

    lax.fori_loop(0, npt // 16, r_step, 0)
    pltpu.sync_copy(res, out_h.at[cid, pl.ds(sid * npt, npt)])


def _agg_body(hsd_h, gidx_h, sidx_h, z_h, out_h, acc,
              gb0, gb1, gb2, gb3, gidxb, sidxb,
              sm0, sm1, sm2, sm3, ig0, ig1, is0, is1):
    cid = lax.axis_index("c")
    sid = lax.axis_index("s")
    n_pad = acc.shape[0]
    npt = n_pad // NS
    gbufs = [gb0, gb1, gb2, gb3]
    sems = [sm0, sm1, sm2, sm3]
    igs = [ig0, ig1]
    iss = [is0, is1]
    nbt = gidx_h.shape[0] // (2 * NS)
    nc = nbt // CH
    pltpu.sync_copy(z_h, acc.at[pl.ds(sid * npt, npt)])
    roff = (cid * NS + sid) * nbt
    plsc.subcore_barrier()

    pltpu.sync_copy(gidx_h.at[pl.ds(roff, CH)], gidxb.at[0])
    pltpu.sync_copy(sidx_h.at[pl.ds(roff, CH)], sidxb.at[0])
    if nc > 1:
        pltpu.async_copy(gidx_h.at[pl.ds(roff + CH, CH)], gidxb.at[1], ig1)
        pltpu.async_copy(sidx_h.at[pl.ds(roff + CH, CH)], sidxb.at[1], is1)
    for b in range(RING):
        pltpu.async_copy(hsd_h.at[gidxb.at[0, b]], gbufs[b], sems[b])

    for c in range(nc):
        s = c % 2
        sn = (c + 1) % 2

        def step(i, c2, s=s):
            for b in range(RING):
                bi = i * RING + b
                pltpu.make_async_copy(hsd_h.at[gidxb.at[s, bi]], gbufs[b],
                                      sems[b]).wait()
                pltpu.sync_copy(gbufs[b], acc.at[sidxb.at[s, bi]], add=True)
                pltpu.async_copy(hsd_h.at[gidxb.at[s, bi + RING]], gbufs[b],
                                 sems[b])
            return c2

        lax.fori_loop(0, CH // RING - 1, step, 0)
        if c + 1 < nc:
            pltpu.make_async_copy(gidx_h.at[pl.ds(roff + (c + 1) * CH, CH)],
                                  gidxb.at[sn], igs[sn]).wait()
            pltpu.make_async_copy(sidx_h.at[pl.ds(roff + (c + 1) * CH, CH)],
                                  sidxb.at[sn], iss[sn]).wait()
            for b in range(RING):
                bi = CH - RING + b
                pltpu.make_async_copy(hsd_h.at[gidxb.at[s, bi]], gbufs[b],
                                      sems[b]).wait()
                pltpu.sync_copy(gbufs[b], acc.at[sidxb.at[s, bi]], add=True)
                pltpu.async_copy(hsd_h.at[gidxb.at[sn, b]], gbufs[b], sems[b])
            if c + 2 < nc:
                pltpu.async_copy(gidx_h.at[pl.ds(roff + (c + 2) * CH, CH)],
                                 gidxb.at[s], igs[s])
                pltpu.async_copy(sidx_h.at[pl.ds(roff + (c + 2) * CH, CH)],
                                 sidxb.at[s], iss[s])
        else:
            for b in range(RING):
                bi = CH - RING + b
                pltpu.make_async_copy(hsd_h.at[gidxb.at[s, bi]], gbufs[b],
                                      sems[b]).wait()
                pltpu.sync_copy(gbufs[b], acc.at[sidxb.at[s, bi]], add=True)
    plsc.subcore_barrier()
    pltpu.sync_copy(acc.at[pl.ds(sid * npt, npt)],
                    out_h.at[cid, pl.ds(sid * npt, npt)])


def _inv_sqrt(deg):
    return jnp.where(deg > 0.0, lax.rsqrt(deg), 0.0)


def _pre_body(x_ref, w_ref, b_ref, deg_ref, h_ref, hsd_ref):
    h = jnp.maximum(
        jnp.dot(x_ref[...], w_ref[...], preferred_element_type=jnp.float32)
        + b_ref[...], 0.0)
    iinv = _inv_sqrt(deg_ref[1, :, 0:1])
    oinv = _inv_sqrt(deg_ref[0, :, 0:1])
    h_ref[...] = h
    hsd_ref[0] = iinv * h
    hsd_ref[1] = oinv * h


def _post_body(agg_ref, deg_ref, h_ref, ws_ref, bs_ref, wd_ref, bd_ref,
               wl_ref, bl_ref, out_ref):
    oinv = _inv_sqrt(deg_ref[0, :, 0:1])
    iinv = _inv_sqrt(deg_ref[1, :, 0:1])
    aggf = agg_ref[0] * oinv
    aggb = agg_ref[1] * iinv
    conv = (0.5 * (jnp.dot(aggf, ws_ref[...],
                           preferred_element_type=jnp.float32) + bs_ref[...])
            + 0.5 * (jnp.dot(aggb, wd_ref[...],
                             preferred_element_type=jnp.float32) + bd_ref[...]))
    h2 = jnp.maximum(conv + h_ref[...], 0.0)
    out_ref[...] = (jnp.dot(h2, wl_ref[...],
                            preferred_element_type=jnp.float32) + bl_ref[...])


@functools.partial(jax.jit, static_argnums=())
def kernel(x, edge_index, W_pre, b_pre, W_s2d, b_s2d, W_d2s, b_d2s,
           W_lin, b_lin):
    n, d = x.shape
    e = edge_index.shape[1]
    npt = (-(-n // NS) + 15) // 16 * 16
    n_pad = npt * NS
    ept = e // NS
    eidx = edge_index.astype(jnp.int32)
    row = eidx[0]
    col = eidx[1]
    gidx = jnp.concatenate([col, row + n])
    sidx = eidx.reshape(2 * e)
    npta = (-(-(n + 1) // NS) + 7) // 8 * 8
    n_pada = npta * NS
    zeros = jnp.zeros((npta, d), jnp.float32)

    mesh = plsc.VectorSubcoreMesh(core_axis_name="c", subcore_axis_name="s")

    deg2 = pl.kernel(
        _deg_body,
        out_type=jax.ShapeDtypeStruct((2, n_pad), jnp.float32),
        mesh=mesh,
        compiler_params=pltpu.CompilerParams(needs_layout_passes=False),
        scratch_types=[
            pltpu.VMEM((n_pad,), jnp.float32),
            pltpu.VMEM((ept,), jnp.int32),
            pltpu.VMEM((NS, npt), jnp.float32),
            pltpu.VMEM((npt,), jnp.float32),
            pltpu.VMEM_SHARED((NS, n_pad), jnp.float32),
        ],
    )(sidx)
    deg3 = deg2[:, :n].reshape(2, n, 1)

    grid = n // ROWS
    h, hsd = pl.pallas_call(
        _pre_body,
        grid=(grid,),
        in_specs=[
            pl.BlockSpec((ROWS, d), lambda i: (i, 0)),
            pl.BlockSpec((d, d), lambda i: (0, 0)),
            pl.BlockSpec((1, d), lambda i: (0, 0)),
            pl.BlockSpec((2, ROWS, 1), lambda i: (0, i, 0)),
        ],
        out_specs=[
            pl.BlockSpec((ROWS, d), lambda i: (i, 0)),
            pl.BlockSpec((2, ROWS, d), lambda i: (0, i, 0)),
        ],
        out_shape=[
            jax.ShapeDtypeStruct((n, d), jnp.float32),
            jax.ShapeDtypeStruct((2, n, d), jnp.float32),
        ],
    )(x, W_pre, b_pre.reshape(1, d), deg3)

    nbt = -(-ept // B)
    nbt = (nbt + CH - 1) // CH * CH
    epp = nbt * B
    gidx2 = jnp.pad(gidx.reshape(2, NS, ept),
                    ((0, 0), (0, 0), (0, epp - ept))).reshape(2 * NS * nbt, B)
    sidx2 = jnp.pad(sidx.reshape(2, NS, ept),
                    ((0, 0), (0, 0), (0, epp - ept)),
                    constant_values=n_pada - 1).reshape(2 * NS * nbt, B)

    agg2 = pl.kernel(
        _agg_body,
        out_type=jax.ShapeDtypeStruct((2, n_pada, d), jnp.float32),
        mesh=mesh,
        scratch_types=[
            pltpu.VMEM_SHARED((n_pada, d), jnp.float32),
            pltpu.VMEM((B, d), jnp.float32),
            pltpu.VMEM((B, d), jnp.float32),
            pltpu.VMEM((B, d), jnp.float32),
            pltpu.VMEM((B, d), jnp.float32),
            pltpu.VMEM((2, CH, B), jnp.int32),
            pltpu.VMEM((2, CH, B), jnp.int32),
            pltpu.SemaphoreType.DMA,
            pltpu.SemaphoreType.DMA,
            pltpu.SemaphoreType.DMA,
            pltpu.SemaphoreType.DMA,
            pltpu.SemaphoreType.DMA,
            pltpu.SemaphoreType.DMA,
            pltpu.SemaphoreType.DMA,
            pltpu.SemaphoreType.DMA,
        ],
    )(hsd.reshape(2 * n, d), gidx2, sidx2, zeros)

    out = pl.pallas_call(
        _post_body,
        grid=(grid,),
        in_specs=[
            pl.BlockSpec((2, ROWS, d), lambda i: (0, i, 0)),
            pl.BlockSpec((2, ROWS, 1), lambda i: (0, i, 0)),
            pl.BlockSpec((ROWS, d), lambda i: (i, 0)),
            pl.BlockSpec((d, d), lambda i: (0, 0)),
            pl.BlockSpec((1, d), lambda i: (0, 0)),
            pl.BlockSpec((d, d), lambda i: (0, 0)),
            pl.BlockSpec((1, d), lambda i: (0, 0)),
            pl.BlockSpec((d, d), lambda i: (0, 0)),
            pl.BlockSpec((1, d), lambda i: (0, 0)),
        ],
        out_specs=pl.BlockSpec((ROWS, d), lambda i: (i, 0)),
        out_shape=jax.ShapeDtypeStruct((n, d), jnp.float32),
    )(agg2, deg3, h, W_s2d, b_s2d.reshape(1, d), W_d2s, b_d2s.reshape(1, d),
      W_lin, b_lin.reshape(1, d))
    return out
